# baseline (speedup 1.0000x reference)
import functools

import jax
import jax.numpy as jnp
import numpy as np
from jax import lax
from jax.experimental import pallas as pl
from jax.experimental.pallas import tpu as pltpu
from jax.experimental.pallas import tpu_sc as plsc

NUM_ENTITY = 1000000
NUM_RELATION = 1000
EMBED_DIM = 64
HALF = EMBED_DIM // 2
MAX_SCORE = 12.0
BATCH = 16384
RELATION_SCALE = float(np.pi) * EMBED_DIM / MAX_SCORE / 2

NC, NS, L = 2, 16, 16
NW = NC * NS
BPW = BATCH // NW
CHUNK = 128
NCHUNK = BPW // CHUNK
GROUPS = BPW // L


def _cs_body(rel_ref, cs_ref):
    r = rel_ref[...] * RELATION_SCALE
    cs_ref[...] = jnp.concatenate([jnp.cos(r), jnp.sin(r)], axis=-1)


_cs_table = pl.pallas_call(
    _cs_body,
    out_shape=jax.ShapeDtypeStruct((NUM_RELATION, EMBED_DIM), jnp.float32),
)


def _sqrt16(x):
    x = jnp.maximum(x, jnp.float32(1e-24))
    i = plsc.bitcast(x, jnp.int32)
    i = jnp.int32(0x5F3759DF) - lax.shift_right_logical(i, 1)
    y = plsc.bitcast(i, jnp.float32)
    xh = x * jnp.float32(-0.5)
    y = y * (jnp.float32(1.5) + xh * y * y)
    y = y * (jnp.float32(1.5) + xh * y * y)
    y = y * (jnp.float32(1.5) + xh * y * y)
    return x * y


_mesh = plsc.VectorSubcoreMesh(core_axis_name="c", subcore_axis_name="s")


@functools.partial(
    pl.kernel,
    out_type=jax.ShapeDtypeStruct((BATCH,), jnp.float32),
    mesh=_mesh,
    compiler_params=pltpu.CompilerParams(
        use_tc_tiling_on_sc=False, needs_layout_passes=False
    ),
    scratch_types=[
        pltpu.VMEM((NCHUNK, CHUNK), jnp.int32),
        pltpu.VMEM((NCHUNK, CHUNK), jnp.int32),
        pltpu.VMEM((NCHUNK, CHUNK), jnp.int32),
        pltpu.VMEM((BPW, EMBED_DIM), jnp.float32),
        pltpu.VMEM((BPW, EMBED_DIM), jnp.float32),
        pltpu.VMEM((BPW, EMBED_DIM), jnp.float32),
        pltpu.VMEM((BPW,), jnp.float32),
        pltpu.SemaphoreType.DMA,
    ],
)
def _sc_score(entity_hbm, cs_hbm, h_hbm, t_hbm, r_hbm, out_hbm,
              hidx, tidx, ridx, hrows, trows, csrows, scores, sem):
    wid = lax.axis_index("s") * NC + lax.axis_index("c")
    row0 = NCHUNK * wid
    pltpu.sync_copy(h_hbm.at[pl.ds(row0, NCHUNK)], hidx)
    pltpu.sync_copy(t_hbm.at[pl.ds(row0, NCHUNK)], tidx)
    pltpu.sync_copy(r_hbm.at[pl.ds(row0, NCHUNK)], ridx)

    copies = []
    for j in range(NCHUNK):
        dst = pl.ds(j * CHUNK, CHUNK)
        copies.append(pltpu.async_copy(entity_hbm.at[hidx.at[j]], hrows.at[dst], sem))
        copies.append(pltpu.async_copy(entity_hbm.at[tidx.at[j]], trows.at[dst], sem))
        copies.append(pltpu.async_copy(cs_hbm.at[ridx.at[j]], csrows.at[dst], sem))
    for c in copies:
        c.wait()

    def group_body(g, carry):
        rows16 = g * L + lax.iota(jnp.int32, L)

        def dim_body(jd, acc):
            col_re = jnp.full((L,), 0, jnp.int32) + jd
            col_im = col_re + HALF
            hre = plsc.load_gather(hrows, [rows16, col_re])
            him = plsc.load_gather(hrows, [rows16, col_im])
            tre = plsc.load_gather(trows, [rows16, col_re])
            tim = plsc.load_gather(trows, [rows16, col_im])
            cc = plsc.load_gather(csrows, [rows16, col_re])
            ss = plsc.load_gather(csrows, [rows16, col_im])
            xre = hre * cc - him * ss - tre
            xim = hre * ss + him * cc - tim
            return acc + _sqrt16(xre * xre + xim * xim)

        acc = lax.fori_loop(0, HALF, dim_body, jnp.zeros((L,), jnp.float32))
        scores[pl.ds(g * L, L)] = jnp.float32(MAX_SCORE) - acc
        return carry

    lax.fori_loop(0, GROUPS, group_body, 0)
    pltpu.sync_copy(scores, out_hbm.at[pl.ds(BPW * wid, BPW)])


def kernel(entity, relation, graph, h_index, t_index, r_index):
    cs = _cs_table(relation)
    h2 = h_index.astype(jnp.int32).reshape(NW * NCHUNK, CHUNK)
    t2 = t_index.astype(jnp.int32).reshape(NW * NCHUNK, CHUNK)
    r2 = r_index.astype(jnp.int32).reshape(NW * NCHUNK, CHUNK)
    return _sc_score(entity, cs, h2, t2, r2)


# speedup vs baseline: 1.1022x; 1.1022x over previous
import functools

import jax
import jax.numpy as jnp
import numpy as np
from jax import lax
from jax.experimental import pallas as pl
from jax.experimental.pallas import tpu as pltpu
from jax.experimental.pallas import tpu_sc as plsc

NUM_ENTITY = 1000000
NUM_RELATION = 1000
EMBED_DIM = 64
HALF = EMBED_DIM // 2
MAX_SCORE = 12.0
BATCH = 16384
RELATION_SCALE = float(np.pi) * EMBED_DIM / MAX_SCORE / 2

NC, NS, L = 2, 16, 16
NW = NC * NS
BPW = BATCH // NW
CHUNK = 128
NCHUNK = BPW // CHUNK
GROUPS = BPW // L


def _cs_body(rel_ref, cs_ref):
    r = rel_ref[...] * RELATION_SCALE
    cs_ref[...] = jnp.concatenate([jnp.cos(r), jnp.sin(r)], axis=-1)


_cs_table = pl.pallas_call(
    _cs_body,
    out_shape=jax.ShapeDtypeStruct((NUM_RELATION, EMBED_DIM), jnp.float32),
)


def _sqrt16(x):
    x = jnp.maximum(x, jnp.float32(1e-24))
    i = plsc.bitcast(x, jnp.int32)
    i = jnp.int32(0x5F3759DF) - lax.shift_right_logical(i, 1)
    y = plsc.bitcast(i, jnp.float32)
    xh = x * jnp.float32(-0.5)
    y = y * (jnp.float32(1.5) + xh * y * y)
    y = y * (jnp.float32(1.5) + xh * y * y)
    y = y * (jnp.float32(1.5) + xh * y * y)
    return x * y


_mesh = plsc.VectorSubcoreMesh(core_axis_name="c", subcore_axis_name="s")


HBPW = BPW // 2


@functools.partial(
    pl.kernel,
    out_type=jax.ShapeDtypeStruct((BATCH,), jnp.float32),
    mesh=_mesh,
    compiler_params=pltpu.CompilerParams(
        use_tc_tiling_on_sc=False, needs_layout_passes=False
    ),
    scratch_types=[
        pltpu.VMEM((NCHUNK, CHUNK), jnp.int32),
        pltpu.VMEM((NCHUNK, CHUNK), jnp.int32),
        pltpu.VMEM((NCHUNK, CHUNK), jnp.int32),
        pltpu.VMEM((HBPW, 128), jnp.float32),
        pltpu.VMEM((HBPW, 128), jnp.float32),
        pltpu.VMEM((HBPW, EMBED_DIM), jnp.float32),
        pltpu.VMEM((BPW,), jnp.float32),
        pltpu.SemaphoreType.DMA,
    ],
)
def _sc_score(entity_hbm, cs_hbm, h_hbm, t_hbm, r_hbm, out_hbm,
              hidx, tidx, ridx, hrows, trows, csrows, scores, sem):
    wid = lax.axis_index("s") * NC + lax.axis_index("c")
    row0 = NCHUNK * wid
    pltpu.sync_copy(h_hbm.at[pl.ds(row0, NCHUNK)], hidx)
    pltpu.sync_copy(t_hbm.at[pl.ds(row0, NCHUNK)], tidx)
    pltpu.sync_copy(r_hbm.at[pl.ds(row0, NCHUNK)], ridx)

    for half in range(2):
        copies = []
        for j in range(NCHUNK // 2):
            cj = half * (NCHUNK // 2) + j
            dst = pl.ds(j * CHUNK, CHUNK)
            copies.append(pltpu.async_copy(
                entity_hbm.at[hidx.at[cj]], hrows.at[dst], sem))
            copies.append(pltpu.async_copy(
                entity_hbm.at[tidx.at[cj]], trows.at[dst], sem))
            copies.append(pltpu.async_copy(
                cs_hbm.at[ridx.at[cj]], csrows.at[dst], sem))
        for c in copies:
            c.wait()

        def group_body(g, carry):
            rows16 = g * L + lax.iota(jnp.int32, L)

            def dim_body(jd, acc):
                col_re = jnp.full((L,), 0, jnp.int32) + jd
                col_im = col_re + HALF
                hre = plsc.load_gather(hrows, [rows16, col_re])
                him = plsc.load_gather(hrows, [rows16, col_im])
                tre = plsc.load_gather(trows, [rows16, col_re])
                tim = plsc.load_gather(trows, [rows16, col_im])
                cc = plsc.load_gather(csrows, [rows16, col_re])
                ss = plsc.load_gather(csrows, [rows16, col_im])
                xre = hre * cc - him * ss - tre
                xim = hre * ss + him * cc - tim
                return acc + _sqrt16(xre * xre + xim * xim)

            acc = lax.fori_loop(0, HALF, dim_body, jnp.zeros((L,), jnp.float32))
            scores[pl.ds(half * HBPW + g * L, L)] = jnp.float32(MAX_SCORE) - acc
            return carry

        lax.fori_loop(0, GROUPS // 2, group_body, 0)

    pltpu.sync_copy(scores, out_hbm.at[pl.ds(BPW * wid, BPW)])


def kernel(entity, relation, graph, h_index, t_index, r_index):
    ep = jnp.pad(entity, ((0, 0), (0, 128 - EMBED_DIM)))
    cs = _cs_table(relation)
    h2 = h_index.astype(jnp.int32).reshape(NW * NCHUNK, CHUNK)
    t2 = t_index.astype(jnp.int32).reshape(NW * NCHUNK, CHUNK)
    r2 = r_index.astype(jnp.int32).reshape(NW * NCHUNK, CHUNK)
    return _sc_score(ep, cs, h2, t2, r2)


# speedup vs baseline: 1.1061x; 1.0036x over previous
import functools

import jax
import jax.numpy as jnp
import numpy as np
from jax import lax
from jax.experimental import pallas as pl
from jax.experimental.pallas import tpu as pltpu
from jax.experimental.pallas import tpu_sc as plsc

NUM_ENTITY = 1000000
NUM_RELATION = 1000
EMBED_DIM = 64
HALF = EMBED_DIM // 2
MAX_SCORE = 12.0
BATCH = 16384
RELATION_SCALE = float(np.pi) * EMBED_DIM / MAX_SCORE / 2

NC, NS, L = 2, 16, 16
NW = NC * NS
BPW = BATCH // NW
CHUNK = 128
NCHUNK = BPW // CHUNK
GROUPS = BPW // L


def _cs_body(rel_ref, cs_ref):
    r = rel_ref[...] * RELATION_SCALE
    cs_ref[...] = jnp.concatenate([jnp.cos(r), jnp.sin(r)], axis=-1)


_cs_table = pl.pallas_call(
    _cs_body,
    out_shape=jax.ShapeDtypeStruct((NUM_RELATION, EMBED_DIM), jnp.float32),
)


def _sqrt16(x):
    x = jnp.maximum(x, jnp.float32(1e-24))
    i = plsc.bitcast(x, jnp.int32)
    i = jnp.int32(0x5F3759DF) - lax.shift_right_logical(i, 1)
    y = plsc.bitcast(i, jnp.float32)
    xh = x * jnp.float32(-0.5)
    y = y * (jnp.float32(1.5) + xh * y * y)
    y = y * (jnp.float32(1.5) + xh * y * y)
    return x * y


_mesh = plsc.VectorSubcoreMesh(core_axis_name="c", subcore_axis_name="s")


HBPW = BPW // 2


@functools.partial(
    pl.kernel,
    out_type=jax.ShapeDtypeStruct((BATCH,), jnp.float32),
    mesh=_mesh,
    compiler_params=pltpu.CompilerParams(
        use_tc_tiling_on_sc=False, needs_layout_passes=False
    ),
    scratch_types=[
        pltpu.VMEM((NCHUNK, CHUNK), jnp.int32),
        pltpu.VMEM((NCHUNK, CHUNK), jnp.int32),
        pltpu.VMEM((NCHUNK, CHUNK), jnp.int32),
        pltpu.VMEM((HBPW, 128), jnp.float32),
        pltpu.VMEM((HBPW, 128), jnp.float32),
        pltpu.VMEM((HBPW, EMBED_DIM), jnp.float32),
        pltpu.VMEM((BPW,), jnp.float32),
        pltpu.SemaphoreType.DMA,
    ],
)
def _sc_score(entity_hbm, cs_hbm, h_hbm, t_hbm, r_hbm, out_hbm,
              hidx, tidx, ridx, hrows, trows, csrows, scores, sem):
    wid = lax.axis_index("s") * NC + lax.axis_index("c")
    row0 = NCHUNK * wid
    pltpu.sync_copy(h_hbm.at[pl.ds(row0, NCHUNK)], hidx)
    pltpu.sync_copy(t_hbm.at[pl.ds(row0, NCHUNK)], tidx)
    pltpu.sync_copy(r_hbm.at[pl.ds(row0, NCHUNK)], ridx)

    for half in range(2):
        copies = []
        for j in range(NCHUNK // 2):
            cj = half * (NCHUNK // 2) + j
            dst = pl.ds(j * CHUNK, CHUNK)
            copies.append(pltpu.async_copy(
                entity_hbm.at[hidx.at[cj]], hrows.at[dst], sem))
            copies.append(pltpu.async_copy(
                entity_hbm.at[tidx.at[cj]], trows.at[dst], sem))
            copies.append(pltpu.async_copy(
                cs_hbm.at[ridx.at[cj]], csrows.at[dst], sem))
        for c in copies:
            c.wait()

        def group_body(g, carry):
            rows16 = g * L + lax.iota(jnp.int32, L)

            def dim_body(jd, acc):
                col_re = jnp.full((L,), 0, jnp.int32) + jd
                col_im = col_re + HALF
                hre = plsc.load_gather(hrows, [rows16, col_re])
                him = plsc.load_gather(hrows, [rows16, col_im])
                tre = plsc.load_gather(trows, [rows16, col_re])
                tim = plsc.load_gather(trows, [rows16, col_im])
                cc = plsc.load_gather(csrows, [rows16, col_re])
                ss = plsc.load_gather(csrows, [rows16, col_im])
                xre = hre * cc - him * ss - tre
                xim = hre * ss + him * cc - tim
                return acc + _sqrt16(xre * xre + xim * xim)

            acc = lax.fori_loop(0, HALF, dim_body, jnp.zeros((L,), jnp.float32))
            scores[pl.ds(half * HBPW + g * L, L)] = jnp.float32(MAX_SCORE) - acc
            return carry

        lax.fori_loop(0, GROUPS // 2, group_body, 0)

    pltpu.sync_copy(scores, out_hbm.at[pl.ds(BPW * wid, BPW)])


def kernel(entity, relation, graph, h_index, t_index, r_index):
    ep = jnp.pad(entity, ((0, 0), (0, 128 - EMBED_DIM)))
    cs = _cs_table(relation)
    h2 = h_index.astype(jnp.int32).reshape(NW * NCHUNK, CHUNK)
    t2 = t_index.astype(jnp.int32).reshape(NW * NCHUNK, CHUNK)
    r2 = r_index.astype(jnp.int32).reshape(NW * NCHUNK, CHUNK)
    return _sc_score(ep, cs, h2, t2, r2)


# speedup vs baseline: 1.2306x; 1.1126x over previous
import functools

import jax
import jax.numpy as jnp
import numpy as np
from jax import lax
from jax.experimental import pallas as pl
from jax.experimental.pallas import tpu as pltpu
from jax.experimental.pallas import tpu_sc as plsc

NUM_ENTITY = 1000000
NUM_RELATION = 1000
EMBED_DIM = 64
HALF = EMBED_DIM // 2
MAX_SCORE = 12.0
BATCH = 16384
RELATION_SCALE = float(np.pi) * EMBED_DIM / MAX_SCORE / 2

NC, NS, L = 2, 16, 16
NW = NC * NS
BPW = BATCH // NW
CHUNK = 128
NCHUNK = BPW // CHUNK
GROUPS = BPW // L


def _cs_body(rel_ref, cs_ref):
    r = rel_ref[...] * RELATION_SCALE
    cs_ref[...] = jnp.concatenate([jnp.cos(r), jnp.sin(r)], axis=-1)


_cs_table = pl.pallas_call(
    _cs_body,
    out_shape=jax.ShapeDtypeStruct((NUM_RELATION, EMBED_DIM), jnp.float32),
)


def _sqrt16(x):
    x = jnp.maximum(x, jnp.float32(1e-24))
    i = plsc.bitcast(x, jnp.int32)
    i = jnp.int32(0x5F3759DF) - lax.shift_right_logical(i, 1)
    y = plsc.bitcast(i, jnp.float32)
    xh = x * jnp.float32(-0.5)
    y = y * (jnp.float32(1.5) + xh * y * y)
    y = y * (jnp.float32(1.5) + xh * y * y)
    return x * y


def _tr_body(in_ref, out_ref):
    xt = in_ref[...].T
    out_ref[...] = jnp.concatenate(
        [xt, jnp.zeros_like(xt)], axis=1)


_TRB = 2048

_tr_call = pl.pallas_call(
    _tr_body,
    grid=(489,),
    in_specs=[pl.BlockSpec((EMBED_DIM, _TRB), lambda i: (0, i))],
    out_specs=pl.BlockSpec((_TRB, 128), lambda i: (i, 0)),
    out_shape=jax.ShapeDtypeStruct((NUM_ENTITY, 128), jnp.float32),
)


_mesh = plsc.VectorSubcoreMesh(core_axis_name="c", subcore_axis_name="s")


HBPW = BPW // 2


@functools.partial(
    pl.kernel,
    out_type=jax.ShapeDtypeStruct((BATCH,), jnp.float32),
    mesh=_mesh,
    compiler_params=pltpu.CompilerParams(
        use_tc_tiling_on_sc=False, needs_layout_passes=False
    ),
    scratch_types=[
        pltpu.VMEM((NCHUNK, CHUNK), jnp.int32),
        pltpu.VMEM((NCHUNK, CHUNK), jnp.int32),
        pltpu.VMEM((NCHUNK, CHUNK), jnp.int32),
        pltpu.VMEM((HBPW, 128), jnp.float32),
        pltpu.VMEM((HBPW, 128), jnp.float32),
        pltpu.VMEM((HBPW, EMBED_DIM), jnp.float32),
        pltpu.VMEM((BPW,), jnp.float32),
        pltpu.SemaphoreType.DMA,
    ],
)
def _sc_score(entity_hbm, cs_hbm, h_hbm, t_hbm, r_hbm, out_hbm,
              hidx, tidx, ridx, hrows, trows, csrows, scores, sem):
    wid = lax.axis_index("s") * NC + lax.axis_index("c")
    row0 = NCHUNK * wid
    pltpu.sync_copy(h_hbm.at[pl.ds(row0, NCHUNK)], hidx)
    pltpu.sync_copy(t_hbm.at[pl.ds(row0, NCHUNK)], tidx)
    pltpu.sync_copy(r_hbm.at[pl.ds(row0, NCHUNK)], ridx)

    for half in range(2):
        copies = []
        for j in range(NCHUNK // 2):
            cj = half * (NCHUNK // 2) + j
            dst = pl.ds(j * CHUNK, CHUNK)
            copies.append(pltpu.async_copy(
                entity_hbm.at[hidx.at[cj]], hrows.at[dst], sem))
            copies.append(pltpu.async_copy(
                entity_hbm.at[tidx.at[cj]], trows.at[dst], sem))
            copies.append(pltpu.async_copy(
                cs_hbm.at[ridx.at[cj]], csrows.at[dst], sem))
        for c in copies:
            c.wait()

        def group_body(g, carry):
            rows16 = g * L + lax.iota(jnp.int32, L)

            def dim_body(jd, acc):
                col_re = jnp.full((L,), 0, jnp.int32) + jd
                col_im = col_re + HALF
                hre = plsc.load_gather(hrows, [rows16, col_re])
                him = plsc.load_gather(hrows, [rows16, col_im])
                tre = plsc.load_gather(trows, [rows16, col_re])
                tim = plsc.load_gather(trows, [rows16, col_im])
                cc = plsc.load_gather(csrows, [rows16, col_re])
                ss = plsc.load_gather(csrows, [rows16, col_im])
                xre = hre * cc - him * ss - tre
                xim = hre * ss + him * cc - tim
                return acc + _sqrt16(xre * xre + xim * xim)

            acc = lax.fori_loop(0, HALF, dim_body, jnp.zeros((L,), jnp.float32))
            scores[pl.ds(half * HBPW + g * L, L)] = jnp.float32(MAX_SCORE) - acc
            return carry

        lax.fori_loop(0, GROUPS // 2, group_body, 0)

    pltpu.sync_copy(scores, out_hbm.at[pl.ds(BPW * wid, BPW)])


def kernel(entity, relation, graph, h_index, t_index, r_index):
    ep = _tr_call(entity.T)
    cs = _cs_table(relation)
    h2 = h_index.astype(jnp.int32).reshape(NW * NCHUNK, CHUNK)
    t2 = t_index.astype(jnp.int32).reshape(NW * NCHUNK, CHUNK)
    r2 = r_index.astype(jnp.int32).reshape(NW * NCHUNK, CHUNK)
    return _sc_score(ep, cs, h2, t2, r2)


# speedup vs baseline: 1.9268x; 1.5658x over previous
import functools

import jax
import jax.numpy as jnp
import numpy as np
from jax import lax
from jax.experimental import pallas as pl
from jax.experimental.pallas import tpu as pltpu
from jax.experimental.pallas import tpu_sc as plsc

NUM_ENTITY = 1000000
NUM_RELATION = 1000
EMBED_DIM = 64
HALF = EMBED_DIM // 2
MAX_SCORE = 12.0
BATCH = 16384
RELATION_SCALE = float(np.pi) * EMBED_DIM / MAX_SCORE / 2

NC, NS, L = 2, 16, 16
NW = NC * NS
BPW = BATCH // NW
CHUNK = 128
NCHUNK = BPW // CHUNK
GROUPS = BPW // L


def _cs_body(rel_ref, cs_ref):
    r = rel_ref[...] * RELATION_SCALE
    cs_ref[...] = jnp.concatenate([jnp.cos(r), jnp.sin(r)], axis=-1)


_cs_table = pl.pallas_call(
    _cs_body,
    out_shape=jax.ShapeDtypeStruct((NUM_RELATION, EMBED_DIM), jnp.float32),
)


def _sqrt16(x):
    x = jnp.maximum(x, jnp.float32(1e-24))
    i = plsc.bitcast(x, jnp.int32)
    i = jnp.int32(0x5F3759DF) - lax.shift_right_logical(i, 1)
    y = plsc.bitcast(i, jnp.float32)
    xh = x * jnp.float32(-0.5)
    y = y * (jnp.float32(1.5) + xh * y * y)
    y = y * (jnp.float32(1.5) + xh * y * y)
    return x * y


def _tr_body(in_ref, out_ref):
    xt = in_ref[...].T
    out_ref[...] = jnp.concatenate(
        [xt, jnp.zeros_like(xt)], axis=1)


_TRB = 8192

_tr_call = pl.pallas_call(
    _tr_body,
    grid=(123,),
    in_specs=[pl.BlockSpec((EMBED_DIM, _TRB), lambda i: (0, i))],
    out_specs=pl.BlockSpec((_TRB, 128), lambda i: (i, 0)),
    out_shape=jax.ShapeDtypeStruct((NUM_ENTITY, 128), jnp.float32),
)


_mesh = plsc.VectorSubcoreMesh(core_axis_name="c", subcore_axis_name="s")


HBPW = BPW // 2


@functools.partial(
    pl.kernel,
    out_type=jax.ShapeDtypeStruct((BATCH,), jnp.float32),
    mesh=_mesh,
    compiler_params=pltpu.CompilerParams(
        use_tc_tiling_on_sc=False, needs_layout_passes=False
    ),
    scratch_types=[
        pltpu.VMEM((NCHUNK, CHUNK), jnp.int32),
        pltpu.VMEM((NCHUNK, CHUNK), jnp.int32),
        pltpu.VMEM((NCHUNK, CHUNK), jnp.int32),
        pltpu.VMEM((HBPW, 128), jnp.float32),
        pltpu.VMEM((HBPW, 128), jnp.float32),
        pltpu.VMEM((HBPW, EMBED_DIM), jnp.float32),
        pltpu.VMEM((BPW,), jnp.float32),
        pltpu.SemaphoreType.DMA,
    ],
)
def _sc_score(entity_hbm, cs_hbm, h_hbm, t_hbm, r_hbm, out_hbm,
              hidx, tidx, ridx, hrows, trows, csrows, scores, sem):
    wid = lax.axis_index("s") * NC + lax.axis_index("c")
    row0 = NCHUNK * wid
    pltpu.sync_copy(h_hbm.at[pl.ds(row0, NCHUNK)], hidx)
    pltpu.sync_copy(t_hbm.at[pl.ds(row0, NCHUNK)], tidx)
    pltpu.sync_copy(r_hbm.at[pl.ds(row0, NCHUNK)], ridx)

    for half in range(2):
        copies = []
        for j in range(NCHUNK // 2):
            cj = half * (NCHUNK // 2) + j
            dst = pl.ds(j * CHUNK, CHUNK)
            copies.append(pltpu.async_copy(
                entity_hbm.at[hidx.at[cj]], hrows.at[dst], sem))
            copies.append(pltpu.async_copy(
                entity_hbm.at[tidx.at[cj]], trows.at[dst], sem))
            copies.append(pltpu.async_copy(
                cs_hbm.at[ridx.at[cj]], csrows.at[dst], sem))
        for c in copies:
            c.wait()

        def group_body(g, carry):
            rows16 = g * L + lax.iota(jnp.int32, L)

            def dim_body(jd, acc):
                col_re = jnp.full((L,), 0, jnp.int32) + jd
                col_im = col_re + HALF
                hre = plsc.load_gather(hrows, [rows16, col_re])
                him = plsc.load_gather(hrows, [rows16, col_im])
                tre = plsc.load_gather(trows, [rows16, col_re])
                tim = plsc.load_gather(trows, [rows16, col_im])
                cc = plsc.load_gather(csrows, [rows16, col_re])
                ss = plsc.load_gather(csrows, [rows16, col_im])
                xre = hre * cc - him * ss - tre
                xim = hre * ss + him * cc - tim
                return acc + _sqrt16(xre * xre + xim * xim)

            acc = lax.fori_loop(0, HALF, dim_body, jnp.zeros((L,), jnp.float32))
            scores[pl.ds(half * HBPW + g * L, L)] = jnp.float32(MAX_SCORE) - acc
            return carry

        lax.fori_loop(0, GROUPS // 2, group_body, 0)

    pltpu.sync_copy(scores, out_hbm.at[pl.ds(BPW * wid, BPW)])


def kernel(entity, relation, graph, h_index, t_index, r_index):
    ep = _tr_call(entity.T)
    cs = _cs_table(relation)
    h2 = h_index.astype(jnp.int32).reshape(NW * NCHUNK, CHUNK)
    t2 = t_index.astype(jnp.int32).reshape(NW * NCHUNK, CHUNK)
    r2 = r_index.astype(jnp.int32).reshape(NW * NCHUNK, CHUNK)
    return _sc_score(ep, cs, h2, t2, r2)


# speedup vs baseline: 2.0412x; 1.0594x over previous
import functools

import jax
import jax.numpy as jnp
import numpy as np
from jax import lax
from jax.experimental import pallas as pl
from jax.experimental.pallas import tpu as pltpu
from jax.experimental.pallas import tpu_sc as plsc

NUM_ENTITY = 1000000
NUM_RELATION = 1000
EMBED_DIM = 64
HALF = EMBED_DIM // 2
MAX_SCORE = 12.0
BATCH = 16384
RELATION_SCALE = float(np.pi) * EMBED_DIM / MAX_SCORE / 2

NC, NS, L = 2, 16, 16
NW = NC * NS
BPW = BATCH // NW
CHUNK = 128
NCHUNK = BPW // CHUNK
GROUPS = BPW // L


def _cs_body(rel_ref, cs_ref):
    r = rel_ref[...] * RELATION_SCALE
    cs_ref[...] = jnp.concatenate([jnp.cos(r), jnp.sin(r)], axis=-1)


_cs_table = pl.pallas_call(
    _cs_body,
    out_shape=jax.ShapeDtypeStruct((NUM_RELATION, EMBED_DIM), jnp.float32),
)


def _sqrt16(x):
    x = jnp.maximum(x, jnp.float32(1e-24))
    i = plsc.bitcast(x, jnp.int32)
    i = jnp.int32(0x5F3759DF) - lax.shift_right_logical(i, 1)
    y = plsc.bitcast(i, jnp.float32)
    xh = x * jnp.float32(-0.5)
    y = y * (jnp.float32(1.5) + xh * y * y)
    y = y * (jnp.float32(1.5) + xh * y * y)
    return x * y


def _tr_body(in_ref, out_ref):
    xt = in_ref[...].T
    out_ref[...] = jnp.concatenate(
        [xt, jnp.zeros_like(xt)], axis=1)


_TRB = 16384

_tr_call = pl.pallas_call(
    _tr_body,
    grid=(62,),
    in_specs=[pl.BlockSpec((EMBED_DIM, _TRB), lambda i: (0, i))],
    out_specs=pl.BlockSpec((_TRB, 128), lambda i: (i, 0)),
    out_shape=jax.ShapeDtypeStruct((NUM_ENTITY, 128), jnp.float32),
)


_mesh = plsc.VectorSubcoreMesh(core_axis_name="c", subcore_axis_name="s")


HBPW = BPW // 2


@functools.partial(
    pl.kernel,
    out_type=jax.ShapeDtypeStruct((BATCH,), jnp.float32),
    mesh=_mesh,
    compiler_params=pltpu.CompilerParams(
        use_tc_tiling_on_sc=False, needs_layout_passes=False
    ),
    scratch_types=[
        pltpu.VMEM((NCHUNK, CHUNK), jnp.int32),
        pltpu.VMEM((NCHUNK, CHUNK), jnp.int32),
        pltpu.VMEM((NCHUNK, CHUNK), jnp.int32),
        pltpu.VMEM((HBPW, 128), jnp.float32),
        pltpu.VMEM((HBPW, 128), jnp.float32),
        pltpu.VMEM((HBPW, EMBED_DIM), jnp.float32),
        pltpu.VMEM((BPW,), jnp.float32),
        pltpu.SemaphoreType.DMA,
    ],
)
def _sc_score(entity_hbm, cs_hbm, h_hbm, t_hbm, r_hbm, out_hbm,
              hidx, tidx, ridx, hrows, trows, csrows, scores, sem):
    wid = lax.axis_index("s") * NC + lax.axis_index("c")
    row0 = NCHUNK * wid
    pltpu.sync_copy(h_hbm.at[pl.ds(row0, NCHUNK)], hidx)
    pltpu.sync_copy(t_hbm.at[pl.ds(row0, NCHUNK)], tidx)
    pltpu.sync_copy(r_hbm.at[pl.ds(row0, NCHUNK)], ridx)

    for half in range(2):
        copies = []
        for j in range(NCHUNK // 2):
            cj = half * (NCHUNK // 2) + j
            dst = pl.ds(j * CHUNK, CHUNK)
            copies.append(pltpu.async_copy(
                entity_hbm.at[hidx.at[cj]], hrows.at[dst], sem))
            copies.append(pltpu.async_copy(
                entity_hbm.at[tidx.at[cj]], trows.at[dst], sem))
            copies.append(pltpu.async_copy(
                cs_hbm.at[ridx.at[cj]], csrows.at[dst], sem))
        for c in copies:
            c.wait()

        def group_body(g, carry):
            rows16 = g * L + lax.iota(jnp.int32, L)

            def dim_body(jd, acc):
                col_re = jnp.full((L,), 0, jnp.int32) + jd
                col_im = col_re + HALF
                hre = plsc.load_gather(hrows, [rows16, col_re])
                him = plsc.load_gather(hrows, [rows16, col_im])
                tre = plsc.load_gather(trows, [rows16, col_re])
                tim = plsc.load_gather(trows, [rows16, col_im])
                cc = plsc.load_gather(csrows, [rows16, col_re])
                ss = plsc.load_gather(csrows, [rows16, col_im])
                xre = hre * cc - him * ss - tre
                xim = hre * ss + him * cc - tim
                return acc + _sqrt16(xre * xre + xim * xim)

            acc = lax.fori_loop(0, HALF, dim_body, jnp.zeros((L,), jnp.float32))
            scores[pl.ds(half * HBPW + g * L, L)] = jnp.float32(MAX_SCORE) - acc
            return carry

        lax.fori_loop(0, GROUPS // 2, group_body, 0)

    pltpu.sync_copy(scores, out_hbm.at[pl.ds(BPW * wid, BPW)])


def kernel(entity, relation, graph, h_index, t_index, r_index):
    ep = _tr_call(entity.T)
    cs = _cs_table(relation)
    h2 = h_index.astype(jnp.int32).reshape(NW * NCHUNK, CHUNK)
    t2 = t_index.astype(jnp.int32).reshape(NW * NCHUNK, CHUNK)
    r2 = r_index.astype(jnp.int32).reshape(NW * NCHUNK, CHUNK)
    return _sc_score(ep, cs, h2, t2, r2)


# speedup vs baseline: 2.1999x; 1.0777x over previous
import functools

import jax
import jax.numpy as jnp
import numpy as np
from jax import lax
from jax.experimental import pallas as pl
from jax.experimental.pallas import tpu as pltpu
from jax.experimental.pallas import tpu_sc as plsc

NUM_ENTITY = 1000000
NUM_RELATION = 1000
EMBED_DIM = 64
HALF = EMBED_DIM // 2
MAX_SCORE = 12.0
BATCH = 16384
RELATION_SCALE = float(np.pi) * EMBED_DIM / MAX_SCORE / 2

NC, NS, L = 2, 16, 16
NW = NC * NS
BPW = BATCH // NW
CHUNK = 128
NCHUNK = BPW // CHUNK
GROUPS = BPW // L


def _cs_body(rel_ref, cs_ref):
    r = rel_ref[...] * RELATION_SCALE
    cs_ref[...] = jnp.concatenate([jnp.cos(r), jnp.sin(r)], axis=-1)


_cs_table = pl.pallas_call(
    _cs_body,
    out_shape=jax.ShapeDtypeStruct((NUM_RELATION, EMBED_DIM), jnp.float32),
)


def _sqrt16(x):
    x = jnp.maximum(x, jnp.float32(1e-24))
    i = plsc.bitcast(x, jnp.int32)
    i = jnp.int32(0x5F3759DF) - lax.shift_right_logical(i, 1)
    y = plsc.bitcast(i, jnp.float32)
    xh = x * jnp.float32(-0.5)
    y = y * (jnp.float32(1.5) + xh * y * y)
    y = y * (jnp.float32(1.5) + xh * y * y)
    return x * y


def _tr_body(inl_ref, inr_ref, out_ref):
    out_ref[...] = jnp.concatenate(
        [inl_ref[...].T, inr_ref[...].T], axis=1)


_TRB = 16384
_SPLIT = 31 * _TRB

_tr_call = pl.pallas_call(
    _tr_body,
    grid=(31,),
    in_specs=[
        pl.BlockSpec((EMBED_DIM, _TRB), lambda i: (0, i)),
        pl.BlockSpec((EMBED_DIM, _TRB), lambda i: (0, i + 31)),
    ],
    out_specs=pl.BlockSpec((_TRB, 128), lambda i: (i, 0)),
    out_shape=jax.ShapeDtypeStruct((_SPLIT, 128), jnp.float32),
)


_mesh = plsc.VectorSubcoreMesh(core_axis_name="c", subcore_axis_name="s")


HBPW = BPW // 2


@functools.partial(
    pl.kernel,
    out_type=jax.ShapeDtypeStruct((BATCH,), jnp.float32),
    mesh=_mesh,
    compiler_params=pltpu.CompilerParams(
        use_tc_tiling_on_sc=False, needs_layout_passes=False
    ),
    scratch_types=[
        pltpu.VMEM((NCHUNK, CHUNK), jnp.int32),
        pltpu.VMEM((NCHUNK, CHUNK), jnp.int32),
        pltpu.VMEM((NCHUNK, CHUNK), jnp.int32),
        pltpu.VMEM((NCHUNK, CHUNK), jnp.int32),
        pltpu.VMEM((NCHUNK, CHUNK), jnp.int32),
        pltpu.VMEM((HBPW, 128), jnp.float32),
        pltpu.VMEM((HBPW, 128), jnp.float32),
        pltpu.VMEM((HBPW, EMBED_DIM), jnp.float32),
        pltpu.VMEM((BPW,), jnp.float32),
        pltpu.SemaphoreType.DMA,
    ],
)
def _sc_score(entity_hbm, cs_hbm, h_hbm, t_hbm, r_hbm, hcb_hbm, tcb_hbm,
              out_hbm, hidx, tidx, ridx, hcbv, tcbv, hrows, trows, csrows,
              scores, sem):
    wid = lax.axis_index("s") * NC + lax.axis_index("c")
    row0 = NCHUNK * wid
    pltpu.sync_copy(h_hbm.at[pl.ds(row0, NCHUNK)], hidx)
    pltpu.sync_copy(t_hbm.at[pl.ds(row0, NCHUNK)], tidx)
    pltpu.sync_copy(r_hbm.at[pl.ds(row0, NCHUNK)], ridx)
    pltpu.sync_copy(hcb_hbm.at[pl.ds(row0, NCHUNK)], hcbv)
    pltpu.sync_copy(tcb_hbm.at[pl.ds(row0, NCHUNK)], tcbv)

    for half in range(2):
        copies = []
        for j in range(NCHUNK // 2):
            cj = half * (NCHUNK // 2) + j
            dst = pl.ds(j * CHUNK, CHUNK)
            copies.append(pltpu.async_copy(
                entity_hbm.at[hidx.at[cj]], hrows.at[dst], sem))
            copies.append(pltpu.async_copy(
                entity_hbm.at[tidx.at[cj]], trows.at[dst], sem))
            copies.append(pltpu.async_copy(
                cs_hbm.at[ridx.at[cj]], csrows.at[dst], sem))
        for c in copies:
            c.wait()

        def group_body(g, carry):
            rows16 = g * L + lax.iota(jnp.int32, L)
            pos16 = half * HBPW + rows16
            hcb = plsc.load_gather(hcbv, [pos16 >> 7, pos16 & 127])
            tcb = plsc.load_gather(tcbv, [pos16 >> 7, pos16 & 127])

            def dim_body(jd, acc):
                col = jnp.full((L,), 0, jnp.int32) + jd
                col_im = col + HALF
                hre = plsc.load_gather(hrows, [rows16, hcb + col])
                him = plsc.load_gather(hrows, [rows16, hcb + col_im])
                tre = plsc.load_gather(trows, [rows16, tcb + col])
                tim = plsc.load_gather(trows, [rows16, tcb + col_im])
                cc = plsc.load_gather(csrows, [rows16, col])
                ss = plsc.load_gather(csrows, [rows16, col_im])
                xre = hre * cc - him * ss - tre
                xim = hre * ss + him * cc - tim
                return acc + _sqrt16(xre * xre + xim * xim)

            acc = lax.fori_loop(0, HALF, dim_body, jnp.zeros((L,), jnp.float32))
            scores[pl.ds(half * HBPW + g * L, L)] = jnp.float32(MAX_SCORE) - acc
            return carry

        lax.fori_loop(0, GROUPS // 2, group_body, 0)

    pltpu.sync_copy(scores, out_hbm.at[pl.ds(BPW * wid, BPW)])


def kernel(entity, relation, graph, h_index, t_index, r_index):
    entT = entity.T
    ep = _tr_call(entT, entT)
    cs = _cs_table(relation)
    h32 = h_index.astype(jnp.int32)
    t32 = t_index.astype(jnp.int32)
    h2 = jnp.where(h32 < _SPLIT, h32, h32 - _SPLIT).reshape(NW * NCHUNK, CHUNK)
    t2 = jnp.where(t32 < _SPLIT, t32, t32 - _SPLIT).reshape(NW * NCHUNK, CHUNK)
    hcb = ((h32 >= _SPLIT).astype(jnp.int32) * EMBED_DIM).reshape(NW * NCHUNK, CHUNK)
    tcb = ((t32 >= _SPLIT).astype(jnp.int32) * EMBED_DIM).reshape(NW * NCHUNK, CHUNK)
    r2 = r_index.astype(jnp.int32).reshape(NW * NCHUNK, CHUNK)
    return _sc_score(ep, cs, h2, t2, r2, hcb, tcb)
